# initial kernel scaffold (unmeasured)
import jax
import jax.numpy as jnp
from jax import lax
from jax.experimental import pallas as pl
from jax.experimental.pallas import tpu as pltpu


def _exchange(logits):
    t, v = logits.shape

    def body(logits_ref, out_ref, copy_sem, send_sem, recv_sem):
        my_x = lax.axis_index("x")
        my_y = lax.axis_index("y")
        my_z = lax.axis_index("z")
        nbr = (my_x, 1 - my_y, my_z)

        barrier_sem = pltpu.get_barrier_semaphore()
        pl.semaphore_signal(
            barrier_sem, inc=1, device_id=nbr,
            device_id_type=pl.DeviceIdType.MESH,
        )
        pl.semaphore_wait(barrier_sem, 1)

        local_cp = pltpu.make_async_copy(
            logits_ref, out_ref.at[:, pl.ds(my_y * v, v)], copy_sem
        )
        local_cp.start()

        rdma = pltpu.make_async_remote_copy(
            src_ref=logits_ref,
            dst_ref=out_ref.at[:, pl.ds(my_y * v, v)],
            send_sem=send_sem,
            recv_sem=recv_sem,
            device_id=nbr,
            device_id_type=pl.DeviceIdType.MESH,
        )
        rdma.start()
        local_cp.wait()
        rdma.wait()

    return pl.pallas_call(
        body,
        out_shape=jax.ShapeDtypeStruct((t, 2 * v), logits.dtype),
        in_specs=[pl.BlockSpec(memory_space=pltpu.ANY)],
        out_specs=pl.BlockSpec(memory_space=pltpu.ANY),
        scratch_shapes=[
            pltpu.SemaphoreType.DMA,
            pltpu.SemaphoreType.DMA,
            pltpu.SemaphoreType.DMA,
        ],
        compiler_params=pltpu.CompilerParams(collective_id=0),
    )(logits)


def kernel(x, W):
    logits = jnp.dot(x, W, preferred_element_type=jnp.float32)
    full = _exchange(logits)
    m = jnp.max(full, axis=-1, keepdims=True)
    e = jnp.exp(full - m)
    return (e / jnp.sum(e, axis=-1, keepdims=True)).astype(jnp.float32)


# baseline (device time: 2294756 ns/iter reference)
import jax
import jax.numpy as jnp
from jax import lax
from jax.experimental import pallas as pl
from jax.experimental.pallas import tpu as pltpu


def _exchange(logits):
    t, v = logits.shape

    def body(logits_ref, out_ref, copy_sem, send_sem, recv_sem):
        my_x = lax.axis_index("x")
        my_y = lax.axis_index("y")
        my_z = lax.axis_index("z")
        nbr = (my_x, 1 - my_y, my_z)

        barrier_sem = pltpu.get_barrier_semaphore()
        pl.semaphore_signal(
            barrier_sem, inc=1, device_id=nbr,
            device_id_type=pl.DeviceIdType.MESH,
        )
        pl.semaphore_wait(barrier_sem, 1)

        local_cp = pltpu.make_async_copy(
            logits_ref, out_ref.at[:, pl.ds(my_y * v, v)], copy_sem
        )
        local_cp.start()

        rdma = pltpu.make_async_remote_copy(
            src_ref=logits_ref,
            dst_ref=out_ref.at[:, pl.ds(my_y * v, v)],
            send_sem=send_sem,
            recv_sem=recv_sem,
            device_id=nbr,
            device_id_type=pl.DeviceIdType.MESH,
        )
        rdma.start()
        local_cp.wait()
        rdma.wait()

    return pl.pallas_call(
        body,
        out_shape=jax.ShapeDtypeStruct((t, 2 * v), logits.dtype),
        in_specs=[pl.BlockSpec(memory_space=pl.ANY)],
        out_specs=pl.BlockSpec(memory_space=pl.ANY),
        scratch_shapes=[
            pltpu.SemaphoreType.DMA,
            pltpu.SemaphoreType.DMA,
            pltpu.SemaphoreType.DMA,
        ],
        compiler_params=pltpu.CompilerParams(collective_id=0),
    )(logits)


def kernel(x, W):
    logits = jnp.dot(x, W, preferred_element_type=jnp.float32)
    full = _exchange(logits)
    m = jnp.max(full, axis=-1, keepdims=True)
    e = jnp.exp(full - m)
    return (e / jnp.sum(e, axis=-1, keepdims=True)).astype(jnp.float32)
